# baseline (speedup 1.0000x reference)
import jax
import jax.numpy as jnp
from jax import lax
from jax.experimental import pallas as pl
from jax.experimental.pallas import tpu as pltpu
from jax.experimental.pallas import tpu_sc as plsc

_N = 8192
_NW = 32
_ROWS = _N // _NW
_CR = 8
_CC = 2048
_NBUF = 5
_CCB = 512
_NT = _ROWS // 16


def _tanh16(v):
    a = jnp.abs(v)
    e = jnp.exp(-2.0 * a)
    t = (1.0 - e) / (1.0 + e)
    return jnp.where(v < 0.0, -t, t)


def _sc_body(x_hbm, o_hbm, buf, dblk, shr, lsem, ssem, dsem, plsem, pssem):
    sid = lax.axis_index("s")
    wid = sid * 2 + lax.axis_index("c")
    base = wid * _ROWS

    def _tile_slice(ref, k):
        return ref.at[
            pl.ds(base + 16 * k, 16),
            pl.ds(base + 128 * (k // 8), 128),
        ]

    dg = [pltpu.async_copy(_tile_slice(x_hbm, k), dblk.at[k], dsem)
          for k in range(_NT)]
    for cp in dg:
        cp.wait()
    ii = lax.iota(jnp.int32, 16)
    masks = [ii == i for i in range(16)]
    for k in range(_NT):
        c16 = 16 * (k % 8)
        rows = [dblk[k, i, pl.ds(c16, 16)] for i in range(16)]
        d = rows[15]
        for i in range(15):
            d = jnp.where(masks[i], rows[i], d)
        t = _tanh16(d)
        for i in range(16):
            dblk[k, i, pl.ds(c16, 16)] = jnp.where(masks[i], t, rows[i])

    n_ch = (_ROWS // _CR) * (_N // _CC)
    n_cc = _N // _CC

    def xs(i):
        return x_hbm.at[pl.ds(base + _CR * (i // n_cc), _CR),
                        pl.ds(_CC * (i % n_cc), _CC)]

    def os(i):
        return o_hbm.at[pl.ds(base + _CR * (i // n_cc), _CR),
                        pl.ds(_CC * (i % n_cc), _CC)]

    def xs_b(g, q):
        return x_hbm.at[pl.ds(base + _CR * g, _CR), pl.ds(_CCB * q, _CCB)]

    def os_b(g, q):
        return o_hbm.at[pl.ds(base + _CR * g, _CR), pl.ds(_CCB * q, _CCB)]

    n_grp = _ROWS // _CR
    groups_b = [g for g in range(n_grp) if g % 4 == 0]
    chunks_a = [i for i in range(n_ch) if (i // n_cc) % 4 != 0]
    chunks_b = [(g, q) for g in groups_b for q in range(_N // _CCB)]
    na, nb = len(chunks_a), len(chunks_b)
    lda = [None] * na
    sta = [None] * na
    ldb = [None] * nb
    stb = [None] * nb
    for i in range(max(na, nb) + 2):
        if i < na:
            if i >= _NBUF:
                sta[i - _NBUF].wait()
            lda[i] = pltpu.async_copy(xs(chunks_a[i]), buf.at[i % _NBUF], lsem)
        if i < nb:
            if i >= 3:
                stb[i - 3].wait()
            ldb[i] = pltpu.async_copy(
                xs_b(*chunks_b[i]), shr.at[sid, i % 3], plsem)
        k = i - 2
        if 0 <= k < na:
            lda[k].wait()
            sta[k] = pltpu.async_copy(buf.at[k % _NBUF], os(chunks_a[k]), ssem)
        if 0 <= k < nb:
            ldb[k].wait()
            stb[k] = pltpu.async_copy(
                shr.at[sid, k % 3], os_b(*chunks_b[k]), pssem)
    for k in range(na - _NBUF, na):
        sta[k].wait()
    for k in range(nb - 3, nb):
        stb[k].wait()

    ds_ = [pltpu.async_copy(dblk.at[k], _tile_slice(o_hbm, k), dsem)
           for k in range(_NT)]
    for cp in ds_:
        cp.wait()


def kernel(x):
    n = x.shape[0]
    mesh = plsc.VectorSubcoreMesh(
        core_axis_name="c", subcore_axis_name="s", num_cores=2, num_subcores=16
    )
    return pl.kernel(
        _sc_body,
        out_type=jax.ShapeDtypeStruct((n, n), x.dtype),
        mesh=mesh,
        scratch_types=[
            pltpu.VMEM((_NBUF, _CR, _CC), jnp.float32),
            pltpu.VMEM((_NT, 16, 128), jnp.float32),
            pltpu.VMEM_SHARED((16, 3, _CR, _CCB), jnp.float32),
            pltpu.SemaphoreType.DMA,
            pltpu.SemaphoreType.DMA,
            pltpu.SemaphoreType.DMA,
            pltpu.SemaphoreType.DMA,
            pltpu.SemaphoreType.DMA,
        ],
    )(x)


# speedup vs baseline: 1.0545x; 1.0545x over previous
import jax
import jax.numpy as jnp
from jax import lax
from jax.experimental import pallas as pl
from jax.experimental.pallas import tpu as pltpu
from jax.experimental.pallas import tpu_sc as plsc

_N = 8192
_NW = 32
_ROWS = _N // _NW
_CR = 8
_CC = 4096
_NT = _ROWS // 16


def _tanh16(v):
    a = jnp.abs(v)
    e = jnp.exp(-2.0 * a)
    t = (1.0 - e) / (1.0 + e)
    return jnp.where(v < 0.0, -t, t)


def _sc_body(x_hbm, o_hbm, buf, dblk, lsem, ssem, dsem):
    wid = lax.axis_index("s") * 2 + lax.axis_index("c")
    base = wid * _ROWS

    def _tile_slice(ref, k):
        return ref.at[
            pl.ds(base + 16 * k, 16),
            pl.ds(base + 128 * (k // 8), 128),
        ]

    dg = [pltpu.async_copy(_tile_slice(x_hbm, k), dblk.at[k], dsem)
          for k in range(_NT)]
    for cp in dg:
        cp.wait()
    ii = lax.iota(jnp.int32, 16)
    masks = [ii == i for i in range(16)]
    for k in range(_NT):
        c16 = 16 * (k % 8)
        rows = [dblk[k, i, pl.ds(c16, 16)] for i in range(16)]
        d = rows[15]
        for i in range(15):
            d = jnp.where(masks[i], rows[i], d)
        t = _tanh16(d)
        for i in range(16):
            dblk[k, i, pl.ds(c16, 16)] = jnp.where(masks[i], t, rows[i])

    n_steps = (_ROWS // _CR) * (_N // _CC) // 2

    def step(j, carry):
        r0 = base + (j // 2) * (2 * _CR)
        r1 = r0 + _CR
        c_lo = (j % 2) * _CC
        c0 = pltpu.async_copy(
            x_hbm.at[pl.ds(r0, _CR), pl.ds(c_lo, _CC)], buf.at[0], lsem)
        c1 = pltpu.async_copy(
            x_hbm.at[pl.ds(r1, _CR), pl.ds(c_lo, _CC)], buf.at[1], lsem)
        c0.wait()
        s0 = pltpu.async_copy(
            buf.at[0], o_hbm.at[pl.ds(r0, _CR), pl.ds(c_lo, _CC)], ssem)
        c1.wait()
        s1 = pltpu.async_copy(
            buf.at[1], o_hbm.at[pl.ds(r1, _CR), pl.ds(c_lo, _CC)], ssem)
        s0.wait()
        s1.wait()
        return carry

    lax.fori_loop(0, n_steps, step, 0)

    ds_ = [pltpu.async_copy(dblk.at[k], _tile_slice(o_hbm, k), dsem)
           for k in range(_NT)]
    for cp in ds_:
        cp.wait()


def kernel(x):
    n = x.shape[0]
    mesh = plsc.VectorSubcoreMesh(
        core_axis_name="c", subcore_axis_name="s", num_cores=2, num_subcores=16
    )
    return pl.kernel(
        _sc_body,
        out_type=jax.ShapeDtypeStruct((n, n), x.dtype),
        mesh=mesh,
        scratch_types=[
            pltpu.VMEM((2, _CR, _CC), jnp.float32),
            pltpu.VMEM((_NT, 16, 128), jnp.float32),
            pltpu.SemaphoreType.DMA,
            pltpu.SemaphoreType.DMA,
            pltpu.SemaphoreType.DMA,
        ],
    )(x)


# speedup vs baseline: 1.0596x; 1.0048x over previous
import jax
import jax.numpy as jnp
from jax import lax
from jax.experimental import pallas as pl
from jax.experimental.pallas import tpu as pltpu
from jax.experimental.pallas import tpu_sc as plsc

_N = 8192
_NW = 32
_ROWS = _N // _NW
_CR = 8
_CC = 4096
_NT = _ROWS // 16


def _tanh16(v):
    a = jnp.abs(v)
    e = jnp.exp(-2.0 * a)
    t = (1.0 - e) / (1.0 + e)
    return jnp.where(v < 0.0, -t, t)


def _sc_body(x_hbm, o_hbm, buf, dblk, lsem, ssem, dsem):
    wid = lax.axis_index("s") * 2 + lax.axis_index("c")
    base = wid * _ROWS

    def _tile_slice(ref, k):
        return ref.at[
            pl.ds(base + 16 * k, 16),
            pl.ds(base + 128 * (k // 8), 128),
        ]

    dg = [pltpu.async_copy(_tile_slice(x_hbm, k), dblk.at[k], dsem)
          for k in range(_NT)]

    n_steps = (_ROWS // _CR) * (_N // _CC) // 2

    def step(j, carry):
        r0 = base + (j // 2) * (2 * _CR)
        r1 = r0 + _CR
        c_lo = (j % 2) * _CC
        c0 = pltpu.async_copy(
            x_hbm.at[pl.ds(r0, _CR), pl.ds(c_lo, _CC)], buf.at[0], lsem)
        c1 = pltpu.async_copy(
            x_hbm.at[pl.ds(r1, _CR), pl.ds(c_lo, _CC)], buf.at[1], lsem)
        c0.wait()
        s0 = pltpu.async_copy(
            buf.at[0], o_hbm.at[pl.ds(r0, _CR), pl.ds(c_lo, _CC)], ssem)
        c1.wait()
        s1 = pltpu.async_copy(
            buf.at[1], o_hbm.at[pl.ds(r1, _CR), pl.ds(c_lo, _CC)], ssem)
        s0.wait()
        s1.wait()
        return carry

    lax.fori_loop(0, n_steps, step, 0)

    for cp in dg:
        cp.wait()
    ii = lax.iota(jnp.int32, 16)
    masks = [ii == i for i in range(16)]
    for k in range(_NT):
        c16 = 16 * (k % 8)
        rows = [dblk[k, i, pl.ds(c16, 16)] for i in range(16)]
        d = rows[15]
        for i in range(15):
            d = jnp.where(masks[i], rows[i], d)
        t = _tanh16(d)
        for i in range(16):
            dblk[k, i, pl.ds(c16, 16)] = jnp.where(masks[i], t, rows[i])

    ds_ = [pltpu.async_copy(dblk.at[k], _tile_slice(o_hbm, k), dsem)
           for k in range(_NT)]
    for cp in ds_:
        cp.wait()


def kernel(x):
    n = x.shape[0]
    mesh = plsc.VectorSubcoreMesh(
        core_axis_name="c", subcore_axis_name="s", num_cores=2, num_subcores=16
    )
    return pl.kernel(
        _sc_body,
        out_type=jax.ShapeDtypeStruct((n, n), x.dtype),
        mesh=mesh,
        scratch_types=[
            pltpu.VMEM((2, _CR, _CC), jnp.float32),
            pltpu.VMEM((_NT, 16, 128), jnp.float32),
            pltpu.SemaphoreType.DMA,
            pltpu.SemaphoreType.DMA,
            pltpu.SemaphoreType.DMA,
        ],
    )(x)
